# Initial kernel scaffold; baseline (speedup 1.0000x reference)
import jax
import jax.numpy as jnp
from jax.experimental import pallas as pl


def kernel(x, edge_index, op_class_id, norm, weight0, w_comp0, weight1, w_comp1, weight2, w_comp2, actor_w1, actor_b1, actor_w2, actor_b2, critic_w1, critic_b1, critic_w2, critic_b2):
    raise NotImplementedError("write your pallas kernel here")


# speedup vs baseline: 10.1362x; 10.1362x over previous
import functools

import jax
import jax.numpy as jnp
from jax import lax
from jax.experimental import pallas as pl
from jax.experimental.pallas import tpu as pltpu
from jax.experimental.pallas import tpu_sc as plsc

_N = 10000
_E = 320000
_R = 16
_B = 8
_D = 128
_T = 400
_NT = _N // _T
_NW = 32
_EPW = _E // _NW
_CH = 80
_NCH = _EPW // _CH
_STRIPE = 640
_NPAD = 16 * _STRIPE
_TRASH = _NPAD - 8


def _tf_body(nparts, use_relu, hp_ref, oh_ref, k_ref, w_ref, y_ref):
    h = hp_ref[0]
    if nparts == 2:
        h = h + hp_ref[1]
    if use_relu:
        h = jnp.maximum(h, 0.0)
    g = jnp.dot(h, k_ref[...], preferred_element_type=jnp.float32)
    acc = jnp.dot(g * oh_ref[:, 0:1], w_ref[0:64, :],
                  preferred_element_type=jnp.float32)
    for r in range(1, _R):
        acc = acc + jnp.dot(g * oh_ref[:, r:r + 1], w_ref[64 * r:64 * r + 64, :],
                            preferred_element_type=jnp.float32)
    y_ref[...] = acc


def _transform(hp, onehotn, k_mat, w2d, use_relu):
    nparts = hp.shape[0]
    return pl.pallas_call(
        functools.partial(_tf_body, nparts, use_relu),
        grid=(_NT,),
        in_specs=[
            pl.BlockSpec((nparts, _T, _D), lambda i: (0, i, 0)),
            pl.BlockSpec((_T, _R), lambda i: (i, 0)),
            pl.BlockSpec((_D, 64), lambda i: (0, 0)),
            pl.BlockSpec((_B * _D, _D), lambda i: (0, 0)),
        ],
        out_specs=pl.BlockSpec((_T, _D), lambda i: (i, 0)),
        out_shape=jax.ShapeDtypeStruct((_N, _D), jnp.float32),
    )(hp, onehotn, k_mat, w2d)


def _agg_body(y_hbm, src_hbm, dst_hbm, z_hbm,
              agg_out, srcb, dstb, rows, aggsh, sem):
    cid = lax.axis_index("c")
    sid = lax.axis_index("s")
    wid = cid * 16 + sid
    pltpu.sync_copy(z_hbm, aggsh.at[pl.ds(sid * _STRIPE, _STRIPE)])
    pltpu.sync_copy(src_hbm.at[wid], srcb)
    pltpu.sync_copy(dst_hbm.at[wid], dstb)
    plsc.subcore_barrier()

    def chunk(c, carry):
        pltpu.async_copy(y_hbm.at[srcb.at[c]], rows, sem).wait()
        pltpu.sync_copy(rows, aggsh.at[dstb.at[c]], add=True)
        return carry

    lax.fori_loop(0, _NCH, chunk, 0)
    plsc.subcore_barrier()
    pltpu.sync_copy(aggsh.at[pl.ds(sid * _STRIPE, _STRIPE)],
                    agg_out.at[cid, pl.ds(sid * _STRIPE, _STRIPE)])


def _sc_edge_agg(y, src3, dst3, z640):
    mesh = plsc.VectorSubcoreMesh(core_axis_name="c", subcore_axis_name="s")
    kern = pl.kernel(
        _agg_body,
        out_type=jax.ShapeDtypeStruct((2, _NPAD, _D), jnp.float32),
        mesh=mesh,
        scratch_types=[
            pltpu.VMEM((_NCH, _CH), jnp.int32),
            pltpu.VMEM((_NCH, _CH), jnp.int32),
            pltpu.VMEM((_CH, _D), jnp.float32),
            pltpu.VMEM_SHARED((_NPAD, _D), jnp.float32),
            pltpu.SemaphoreType.DMA,
        ],
    )
    return kern(y, src3, dst3, z640)


def _cnt_body(idx_hbm, zc_hbm, ones_hbm,
              cnt_out, idxb, onesb, cntsh, sem):
    cid = lax.axis_index("c")
    sid = lax.axis_index("s")
    wid = cid * 16 + sid
    pltpu.sync_copy(zc_hbm, cntsh.at[pl.ds(sid * _STRIPE, _STRIPE)])
    pltpu.sync_copy(ones_hbm, onesb)
    pltpu.sync_copy(idx_hbm.at[wid], idxb)
    plsc.subcore_barrier()

    def chunk(c, carry):
        pltpu.sync_copy(onesb, cntsh.at[idxb.at[c]], add=True)
        return carry

    lax.fori_loop(0, _NCH, chunk, 0)
    plsc.subcore_barrier()
    pltpu.sync_copy(cntsh.at[pl.ds(sid * _STRIPE, _STRIPE)],
                    cnt_out.at[cid, pl.ds(sid * _STRIPE, _STRIPE)])


def _sc_count(idx3, zc640, ones80):
    mesh = plsc.VectorSubcoreMesh(core_axis_name="c", subcore_axis_name="s")
    kern = pl.kernel(
        _cnt_body,
        out_type=jax.ShapeDtypeStruct((2, _NPAD, _D), jnp.float32),
        mesh=mesh,
        scratch_types=[
            pltpu.VMEM((_NCH, _CH), jnp.int32),
            pltpu.VMEM((_CH, _D), jnp.float32),
            pltpu.VMEM_SHARED((_NPAD, _D), jnp.float32),
            pltpu.SemaphoreType.DMA,
        ],
    )
    return kern(idx3, zc640, ones80)


def _final_body(agg_ref, cnt_ref, ohn_ref, k_ref, w_ref,
                aw1_ref, ab1_ref, aw2_ref, ab2_ref,
                cw1_ref, cb1_ref, cw2_ref, cb2_ref,
                probs_ref, val_ref, s_ref):
    i = pl.program_id(0)
    h2 = jnp.maximum(agg_ref[0] + agg_ref[1], 0.0)
    g2 = jnp.dot(h2, k_ref[...], preferred_element_type=jnp.float32)
    c0 = (cnt_ref[0] + cnt_ref[1])[:, 0:1]
    lhs = ohn_ref[...] * c0
    contrib = lax.dot_general(lhs, g2, (((0,), (0,)), ((), ())),
                              preferred_element_type=jnp.float32)

    @pl.when(i == 0)
    def _():
        s_ref[...] = contrib

    @pl.when(i > 0)
    def _():
        s_ref[...] = s_ref[...] + contrib

    @pl.when(i == pl.num_programs(0) - 1)
    def _():
        s = s_ref[...]
        root = jnp.dot(s[0:1, :], w_ref[0:64, :],
                       preferred_element_type=jnp.float32)
        for r in range(1, _R):
            root = root + jnp.dot(s[r:r + 1, :], w_ref[64 * r:64 * r + 64, :],
                                  preferred_element_type=jnp.float32)
        root = root - jnp.max(root)
        e = jnp.exp(root)
        p = e / jnp.sum(e)
        ha = jnp.maximum(jnp.dot(p, aw1_ref[...],
                                 preferred_element_type=jnp.float32)
                         + ab1_ref[...], 0.0)
        probs_ref[...] = jnp.dot(ha, aw2_ref[...],
                                 preferred_element_type=jnp.float32) + ab2_ref[...]
        hc = jnp.maximum(jnp.dot(p, cw1_ref[...],
                                 preferred_element_type=jnp.float32)
                         + cb1_ref[...], 0.0)
        val_ref[...] = (jnp.sum(hc * cw2_ref[...], axis=1, keepdims=True)
                        + cb2_ref[...])


def _final(agg2, cnt, onehotn, k_mat, w2d,
           aw1t, ab1, aw2t, ab2, cw1t, cb1, cw2, cb2):
    return pl.pallas_call(
        _final_body,
        grid=(_NT,),
        in_specs=[
            pl.BlockSpec((2, _T, _D), lambda i: (0, i, 0)),
            pl.BlockSpec((2, _T, _D), lambda i: (0, i, 0)),
            pl.BlockSpec((_T, _R), lambda i: (i, 0)),
            pl.BlockSpec((_D, 64), lambda i: (0, 0)),
            pl.BlockSpec((_B * _D, _D), lambda i: (0, 0)),
            pl.BlockSpec((_D, 256), lambda i: (0, 0)),
            pl.BlockSpec((1, 256), lambda i: (0, 0)),
            pl.BlockSpec((256, _D), lambda i: (0, 0)),
            pl.BlockSpec((1, _D), lambda i: (0, 0)),
            pl.BlockSpec((_D, 256), lambda i: (0, 0)),
            pl.BlockSpec((1, 256), lambda i: (0, 0)),
            pl.BlockSpec((1, 256), lambda i: (0, 0)),
            pl.BlockSpec((1, 1), lambda i: (0, 0)),
        ],
        out_specs=[pl.BlockSpec((1, _D), lambda i: (0, 0)),
                   pl.BlockSpec((1, 1), lambda i: (0, 0))],
        out_shape=[jax.ShapeDtypeStruct((1, _D), jnp.float32),
                   jax.ShapeDtypeStruct((1, 1), jnp.float32)],
        scratch_shapes=[pltpu.VMEM((_R, 64), jnp.float32)],
    )(agg2, cnt, onehotn, k_mat, w2d,
      aw1t, ab1, aw2t, ab2, cw1t, cb1, cw2, cb2)


def kernel(x, edge_index, op_class_id, norm,
           weight0, w_comp0, weight1, w_comp1, weight2, w_comp2,
           actor_w1, actor_b1, actor_w2, actor_b2,
           critic_w1, critic_b1, critic_w2, critic_b2):
    src_flat = edge_index[0].astype(jnp.int32)
    dst_flat = edge_index[1].astype(jnp.int32)
    src = src_flat.reshape(_NW, _NCH, _CH)
    dst = dst_flat.reshape(_NW, _NCH, _CH)
    idx3 = jnp.where(dst_flat == 0, src_flat, _TRASH).reshape(_NW, _NCH, _CH)

    eye8 = jnp.eye(_B, dtype=jnp.float32)
    k0 = jnp.kron(eye8, w_comp0)
    k1 = jnp.kron(eye8, w_comp1)
    k2 = jnp.kron(eye8, w_comp2)
    w0 = weight0.reshape(_B * _D, _D)
    w1 = weight1.reshape(_B * _D, _D)
    w2 = weight2.reshape(_B * _D, _D)

    rel = op_class_id.astype(jnp.int32)
    onehot = (rel[:, None] == jnp.arange(_R, dtype=jnp.int32)[None, :])
    onehotn = onehot.astype(jnp.float32) * norm[:, None]

    z640 = jnp.zeros((_STRIPE, _D), jnp.float32)
    ones80 = jnp.ones((_CH, _D), jnp.float32)

    cnt = _sc_count(idx3, z640, ones80)
    y0 = _transform(x[None], onehotn, k0, w0, use_relu=False)
    agg1 = _sc_edge_agg(y0, src, dst, z640)
    y1 = _transform(agg1, onehotn, k1, w1, use_relu=True)
    agg2 = _sc_edge_agg(y1, src, dst, z640)

    probs2, val2 = _final(
        agg2, cnt, onehotn, k2, w2,
        actor_w1.T, actor_b1.reshape(1, 256), actor_w2.T,
        actor_b2.reshape(1, _D),
        critic_w1.T, critic_b1.reshape(1, 256),
        critic_w2.reshape(1, 256), critic_b2.reshape(1, 1))
    return probs2.reshape(_D), val2.reshape(1)


# speedup vs baseline: 14.8483x; 1.4649x over previous
import functools

import jax
import jax.numpy as jnp
from jax import lax
from jax.experimental import pallas as pl
from jax.experimental.pallas import tpu as pltpu
from jax.experimental.pallas import tpu_sc as plsc

_N = 10000
_E = 320000
_R = 16
_B = 8
_D = 128
_T = 400
_NT = _N // _T
_NW = 32
_EPW = _E // _NW
_CH = 125
_NCH = _EPW // _CH
_STRIPE = 640
_NPAD = 16 * _STRIPE
_TRASH = _NPAD - 8


def _tf_body(nparts, use_relu, hp_ref, oh_ref, k_ref, w_ref, y_ref):
    h = hp_ref[0]
    if nparts == 2:
        h = h + hp_ref[1]
    if use_relu:
        h = jnp.maximum(h, 0.0)
    g = jnp.dot(h, k_ref[...], preferred_element_type=jnp.float32)
    acc = jnp.dot(g * oh_ref[:, 0:1], w_ref[0:64, :],
                  preferred_element_type=jnp.float32)
    for r in range(1, _R):
        acc = acc + jnp.dot(g * oh_ref[:, r:r + 1], w_ref[64 * r:64 * r + 64, :],
                            preferred_element_type=jnp.float32)
    y_ref[...] = acc


def _transform(hp, onehotn, k_mat, w2d, use_relu):
    nparts = hp.shape[0]
    return pl.pallas_call(
        functools.partial(_tf_body, nparts, use_relu),
        grid=(_NT,),
        in_specs=[
            pl.BlockSpec((nparts, _T, _D), lambda i: (0, i, 0)),
            pl.BlockSpec((_T, _R), lambda i: (i, 0)),
            pl.BlockSpec((_D, 64), lambda i: (0, 0)),
            pl.BlockSpec((_B * _D, _D), lambda i: (0, 0)),
        ],
        out_specs=pl.BlockSpec((_T, _D), lambda i: (i, 0)),
        out_shape=jax.ShapeDtypeStruct((_N, _D), jnp.float32),
    )(hp, onehotn, k_mat, w2d)


def _agg_body(y_hbm, src_hbm, dst_hbm, z_hbm,
              agg_out, srcb, dstb, rows0, aggsh, sem0):
    cid = lax.axis_index("c")
    sid = lax.axis_index("s")
    wid = cid * 16 + sid
    pltpu.sync_copy(z_hbm, aggsh.at[pl.ds(sid * _STRIPE, _STRIPE)])
    pltpu.sync_copy(src_hbm.at[wid], srcb)
    pltpu.sync_copy(dst_hbm.at[wid], dstb)
    plsc.subcore_barrier()

    def chunk(c, carry):
        pltpu.async_copy(y_hbm.at[srcb.at[c]], rows0, sem0).wait()
        pltpu.sync_copy(rows0, aggsh.at[dstb.at[c]], add=True)
        return carry

    lax.fori_loop(0, _NCH, chunk, 0)
    plsc.subcore_barrier()
    pltpu.sync_copy(aggsh.at[pl.ds(sid * _STRIPE, _STRIPE)],
                    agg_out.at[cid, pl.ds(sid * _STRIPE, _STRIPE)])


def _sc_edge_agg(y, src3, dst3, z640):
    mesh = plsc.VectorSubcoreMesh(core_axis_name="c", subcore_axis_name="s")
    kern = pl.kernel(
        _agg_body,
        out_type=jax.ShapeDtypeStruct((2, _NPAD, _D), jnp.float32),
        mesh=mesh,
        scratch_types=[
            pltpu.VMEM((_NCH, _CH), jnp.int32),
            pltpu.VMEM((_NCH, _CH), jnp.int32),
            pltpu.VMEM((_CH, _D), jnp.float32),
            pltpu.VMEM_SHARED((_NPAD, _D), jnp.float32),
            pltpu.SemaphoreType.DMA,
        ],
    )
    return kern(y, src3, dst3, z640)


def _cnt_body(idx_hbm, flag_hbm, zc_hbm, ones_hbm,
              cnt_out, idxb, flagb, onesb, cntsh, sem):
    cid = lax.axis_index("c")
    sid = lax.axis_index("s")
    wid = cid * 16 + sid
    pltpu.sync_copy(zc_hbm, cntsh.at[pl.ds(sid * _STRIPE, _STRIPE)])
    pltpu.sync_copy(ones_hbm, onesb)
    pltpu.sync_copy(idx_hbm.at[wid], idxb)
    pltpu.sync_copy(flag_hbm.at[wid], flagb)
    plsc.subcore_barrier()

    def chunk(c, carry):
        fv = flagb[pl.ds(c, 16)]

        @pl.when(fv[0] != 0)
        def _():
            pltpu.sync_copy(onesb, cntsh.at[idxb.at[c]], add=True)
        return carry

    lax.fori_loop(0, _NCH, chunk, 0)
    plsc.subcore_barrier()
    pltpu.sync_copy(cntsh.at[pl.ds(sid * _STRIPE, _STRIPE)],
                    cnt_out.at[cid, pl.ds(sid * _STRIPE, _STRIPE)])


def _sc_count(idx3, flags, zc640, ones80):
    mesh = plsc.VectorSubcoreMesh(core_axis_name="c", subcore_axis_name="s")
    kern = pl.kernel(
        _cnt_body,
        out_type=jax.ShapeDtypeStruct((2, _NPAD, _D), jnp.float32),
        mesh=mesh,
        scratch_types=[
            pltpu.VMEM((_NCH, _CH), jnp.int32),
            pltpu.VMEM((_NCH + 16,), jnp.int32),
            pltpu.VMEM((_CH, _D), jnp.float32),
            pltpu.VMEM_SHARED((_NPAD, _D), jnp.float32),
            pltpu.SemaphoreType.DMA,
        ],
    )
    return kern(idx3, flags, zc640, ones80)


def _final_body(agg_ref, cnt_ref, ohn_ref, k_ref, w_ref,
                aw1_ref, ab1_ref, aw2_ref, ab2_ref,
                cw1_ref, cb1_ref, cw2_ref, cb2_ref,
                probs_ref, val_ref, s_ref):
    i = pl.program_id(0)
    h2 = jnp.maximum(agg_ref[0] + agg_ref[1], 0.0)
    g2 = jnp.dot(h2, k_ref[...], preferred_element_type=jnp.float32)
    c0 = (cnt_ref[0] + cnt_ref[1])[:, 0:1]
    lhs = ohn_ref[...] * c0
    contrib = lax.dot_general(lhs, g2, (((0,), (0,)), ((), ())),
                              preferred_element_type=jnp.float32)

    @pl.when(i == 0)
    def _():
        s_ref[...] = contrib

    @pl.when(i > 0)
    def _():
        s_ref[...] = s_ref[...] + contrib

    @pl.when(i == pl.num_programs(0) - 1)
    def _():
        s = s_ref[...]
        root = jnp.dot(s[0:1, :], w_ref[0:64, :],
                       preferred_element_type=jnp.float32)
        for r in range(1, _R):
            root = root + jnp.dot(s[r:r + 1, :], w_ref[64 * r:64 * r + 64, :],
                                  preferred_element_type=jnp.float32)
        root = root - jnp.max(root)
        e = jnp.exp(root)
        p = e / jnp.sum(e)
        ha = jnp.maximum(jnp.dot(p, aw1_ref[...],
                                 preferred_element_type=jnp.float32)
                         + ab1_ref[...], 0.0)
        probs_ref[...] = jnp.dot(ha, aw2_ref[...],
                                 preferred_element_type=jnp.float32) + ab2_ref[...]
        hc = jnp.maximum(jnp.dot(p, cw1_ref[...],
                                 preferred_element_type=jnp.float32)
                         + cb1_ref[...], 0.0)
        val_ref[...] = (jnp.sum(hc * cw2_ref[...], axis=1, keepdims=True)
                        + cb2_ref[...])


def _final(agg2, cnt, onehotn, k_mat, w2d,
           aw1t, ab1, aw2t, ab2, cw1t, cb1, cw2, cb2):
    return pl.pallas_call(
        _final_body,
        grid=(_NT,),
        in_specs=[
            pl.BlockSpec((2, _T, _D), lambda i: (0, i, 0)),
            pl.BlockSpec((2, _T, _D), lambda i: (0, i, 0)),
            pl.BlockSpec((_T, _R), lambda i: (i, 0)),
            pl.BlockSpec((_D, 64), lambda i: (0, 0)),
            pl.BlockSpec((_B * _D, _D), lambda i: (0, 0)),
            pl.BlockSpec((_D, 256), lambda i: (0, 0)),
            pl.BlockSpec((1, 256), lambda i: (0, 0)),
            pl.BlockSpec((256, _D), lambda i: (0, 0)),
            pl.BlockSpec((1, _D), lambda i: (0, 0)),
            pl.BlockSpec((_D, 256), lambda i: (0, 0)),
            pl.BlockSpec((1, 256), lambda i: (0, 0)),
            pl.BlockSpec((1, 256), lambda i: (0, 0)),
            pl.BlockSpec((1, 1), lambda i: (0, 0)),
        ],
        out_specs=[pl.BlockSpec((1, _D), lambda i: (0, 0)),
                   pl.BlockSpec((1, 1), lambda i: (0, 0))],
        out_shape=[jax.ShapeDtypeStruct((1, _D), jnp.float32),
                   jax.ShapeDtypeStruct((1, 1), jnp.float32)],
        scratch_shapes=[pltpu.VMEM((_R, 64), jnp.float32)],
    )(agg2, cnt, onehotn, k_mat, w2d,
      aw1t, ab1, aw2t, ab2, cw1t, cb1, cw2, cb2)


def kernel(x, edge_index, op_class_id, norm,
           weight0, w_comp0, weight1, w_comp1, weight2, w_comp2,
           actor_w1, actor_b1, actor_w2, actor_b2,
           critic_w1, critic_b1, critic_w2, critic_b2):
    src_flat = edge_index[0].astype(jnp.int32)
    dst_flat = edge_index[1].astype(jnp.int32)
    src = src_flat.reshape(_NW, _NCH, _CH)
    dst = dst_flat.reshape(_NW, _NCH, _CH)
    idx3 = jnp.where(dst_flat == 0, src_flat, _TRASH).reshape(_NW, _NCH, _CH)
    flags = jnp.any(dst.reshape(_NW, _NCH, _CH) == 0, axis=2).astype(jnp.int32)
    flags = jnp.pad(flags, ((0, 0), (0, 16)))

    eye8 = jnp.eye(_B, dtype=jnp.float32)
    k0 = jnp.kron(eye8, w_comp0)
    k1 = jnp.kron(eye8, w_comp1)
    k2 = jnp.kron(eye8, w_comp2)
    w0 = weight0.reshape(_B * _D, _D)
    w1 = weight1.reshape(_B * _D, _D)
    w2 = weight2.reshape(_B * _D, _D)

    rel = op_class_id.astype(jnp.int32)
    onehot = (rel[:, None] == jnp.arange(_R, dtype=jnp.int32)[None, :])
    onehotn = onehot.astype(jnp.float32) * norm[:, None]

    z640 = jnp.zeros((_STRIPE, _D), jnp.float32)
    ones80 = jnp.ones((_CH, _D), jnp.float32)

    cnt = _sc_count(idx3, flags, z640, ones80)
    y0 = _transform(x[None], onehotn, k0, w0, use_relu=False)
    agg1 = _sc_edge_agg(y0, src, dst, z640)
    y1 = _transform(agg1, onehotn, k1, w1, use_relu=True)
    agg2 = _sc_edge_agg(y1, src, dst, z640)

    probs2, val2 = _final(
        agg2, cnt, onehotn, k2, w2,
        actor_w1.T, actor_b1.reshape(1, 256), actor_w2.T,
        actor_b2.reshape(1, _D),
        critic_w1.T, critic_b1.reshape(1, 256),
        critic_w2.reshape(1, 256), critic_b2.reshape(1, 1))
    return probs2.reshape(_D), val2.reshape(1)
